# baseline (speedup 1.0000x reference)
import jax
import jax.numpy as jnp
from jax import lax
from jax.experimental import pallas as pl
from jax.experimental.pallas import tpu as pltpu

N_TREES_C = 1000
NPT = 50
CPT = NPT - 1
HID = 128
LAT = 56
VOC = 780
TPB = 4
NB = N_TREES_C // TPB
NBLK = NPT * TPB
PB = CPT * TPB


def _sig(x):
    return 1.0 / (1.0 + jnp.exp(-x))


def _dot(a, b):
    return jnp.dot(a, b, preferred_element_type=jnp.float32)


def _dot_t(a, b):
    return lax.dot_general(a, b, (((0,), (0,)), ((), ())),
                           preferred_element_type=jnp.float32)


def _tables_kernel(emb_ref, wz_ref, bz_ref, wr_ref, br_ref,
                   wh_ref, bh_ref, uw_ref, bu_ref, out_ref):
    e = emb_ref[...]
    cz = _dot(e, wz_ref[0:HID, :]) + bz_ref[...]
    ch = _dot(e, wh_ref[0:HID, :]) + bh_ref[...]
    m1 = _sig(cz) * jnp.tanh(ch)
    ar = _dot(e, wr_ref[...]) + br_ref[...]
    xu = _dot(e, uw_ref[0:HID, :]) + bu_ref[...]
    out_ref[...] = jnp.concatenate([m1, cz, ch, ar, xu], axis=1)


def _main_kernel(wid_ref, par_ref, pt_ref, tv_ref, t5_ref,
                 ur_ref, wh2_ref, wz2_ref, wqh_ref, wqt_ref, bw_ref, wo_ref,
                 bo_ref, uh_ref, ut_ref, us_ref, bs_ref, out_ref):
    f32 = jnp.float32
    wid = wid_ref[0, 0, :]
    wid_col = wid.reshape(NBLK, 1)
    iota_v = lax.broadcasted_iota(jnp.int32, (NBLK, VOC), 1)
    o_w = (iota_v == wid_col).astype(f32)
    nt = _dot(o_w, t5_ref[...])
    n_m1 = nt[:, 0:128]
    n_cz = nt[:, 128:256]
    n_ch = nt[:, 256:384]
    n_ar = nt[:, 384:512]
    n_xu = nt[:, 512:640]
    n_mz = _dot(n_m1, wz2_ref[...])
    n_br = _dot(n_m1, ur_ref[...])

    par_col = par_ref[0, 0, :].reshape(PB, 1)
    iota_pn = lax.broadcasted_iota(jnp.int32, (PB, NBLK), 1)
    o_par = (iota_pn == par_col).astype(f32)

    ntp = jnp.concatenate([n_m1, n_cz, n_ch, n_ar], axis=1)
    pt4 = _dot(o_par, ntp)
    p_m1 = pt4[:, 0:128]
    p_cz = pt4[:, 128:256]
    p_ch = pt4[:, 256:384]
    p_ar = pt4[:, 384:512]
    p_mz = _dot(p_m1, wz2_ref[...])
    p_br = _dot(p_m1, ur_ref[...])
    c_m1 = n_m1[0:PB, :]
    c_mz = n_mz[0:PB, :]
    c_cz = n_cz[0:PB, :]
    c_ch = n_ch[0:PB, :]
    c_ar = n_ar[0:PB, :]
    c_br = n_br[0:PB, :]

    rm1_e = _sig(c_ar + p_br) * p_m1
    rm1_o = _sig(p_ar + c_br) * c_m1
    g_eo = _dot(jnp.concatenate([rm1_e, rm1_o], axis=0), wh2_ref[...])
    g_e = g_eo[0:PB, :]
    g_o = g_eo[PB:2 * PB, :]

    zpad = jnp.zeros((NBLK - PB, 2 * HID), f32)
    sn2 = (jnp.concatenate([jnp.concatenate([p_m1, g_e], axis=1), zpad],
                           axis=0)
           + _dot_t(o_par, jnp.concatenate([c_m1, g_o], axis=1)))
    sn_m1 = sn2[:, 0:128]
    sn_g = sn2[:, 128:256]
    sn_mz = _dot(sn_m1, wz2_ref[...])
    s_e = _dot(o_par, sn2)
    s_mz_e = _dot(s_e[:, 0:128], wz2_ref[...])
    s_o = jnp.concatenate([sn_m1, sn_mz, sn_g], axis=1)[0:PB, :]

    z2_e = _sig(p_cz + s_mz_e - c_mz)
    t2_e = jnp.tanh(p_ch + s_e[:, 128:256] - g_o)
    s2_e = s_e[:, 0:128] - c_m1
    m2_e = (1.0 - z2_e) * s2_e + z2_e * t2_e
    z2_o = _sig(c_cz + s_o[:, 128:256] - p_mz)
    t2_o = jnp.tanh(c_ch + s_o[:, 256:384] - g_e)
    s2_o = s_o[:, 0:128] - p_m1
    m2_o = (1.0 - z2_o) * s2_o + z2_o * t2_o

    zpad1 = jnp.zeros((NBLK - PB, HID), f32)
    h = (jnp.concatenate([m2_e, zpad1], axis=0)
         + _dot_t(o_par, m2_o))

    tv8 = tv_ref[0]
    jj = lax.broadcasted_iota(jnp.int32, (NBLK, TPB), 0)
    t_of = jnp.where(jj < PB, jj // CPT, jj - PB)
    o_t = (t_of == lax.broadcasted_iota(jnp.int32, (NBLK, TPB), 1)).astype(f32)
    tvn = _dot(o_t, tv8)

    qpre = jnp.maximum(_dot(h, wqh_ref[...]) + _dot(tvn, wqt_ref[...])
                       + bw_ref[...], 0.0)
    q = _dot(qpre, wo_ref[...]) + bo_ref[...]
    mx = jnp.max(q, axis=1, keepdims=True)
    lse = jnp.log(jnp.sum(jnp.exp(q - mx), axis=1, keepdims=True)) + mx
    qtrue = jnp.sum(q * o_w, axis=1, keepdims=True)
    qlp = jnp.sum(lse - qtrue)
    idxmat = jnp.where(q == mx, iota_v, VOC)
    amax = jnp.min(idxmat, axis=1, keepdims=True)
    qacc = jnp.sum((amax == wid_col).astype(f32))

    ppre = jnp.maximum(n_xu + _dot(h, uh_ref[...]) + _dot(tvn, ut_ref[...]),
                       0.0)
    p = _dot(ppre, us_ref[...]) + bs_ref[0:1, 0:1]
    ptg = pt_ref[0, 0, :].reshape(NBLK, 1)
    plp = jnp.sum(jnp.maximum(p, 0.0) - p * ptg
                  + jnp.log(1.0 + jnp.exp(-jnp.abs(p))))
    pacc = jnp.sum(((p > 0.0) == (ptg > 0.5)).astype(f32))

    li = lax.broadcasted_iota(jnp.int32, (8, 128), 1)
    si = lax.broadcasted_iota(jnp.int32, (8, 128), 0)
    row0 = si == 0
    res = (jnp.where(row0 & (li == 0), qlp, 0.0)
           + jnp.where(row0 & (li == 1), plp, 0.0)
           + jnp.where(row0 & (li == 2), pacc, 0.0)
           + jnp.where(row0 & (li == 3), qacc, 0.0))
    out_ref[0] = res


def kernel(wid, edge_src, edge_dst, rev_edge, node2tree, p_targets, tree_vec,
           emb, W_z, b_z, W_r, U_r, b_r, W_h, b_h, W_w, b_w, U_w, b_u,
           Wo, b_o, Us, b_s):
    f32 = jnp.float32
    i32 = jnp.int32

    jj = jnp.arange(NBLK, dtype=i32)
    t_in = jnp.where(jj < PB, jj // CPT, jj - PB)
    c_of = jnp.where(jj < PB, jj % CPT + 1, 0)
    old_local = t_in * NPT + c_of

    wid3 = wid.astype(i32).reshape(NB, NBLK)[:, old_local].reshape(NB, 1, NBLK)
    ptgt = (p_targets.astype(f32).reshape(NB, NBLK)[:, old_local]
            .reshape(NB, 1, NBLK))

    par_old = edge_src.astype(i32).reshape(N_TREES_C, CPT, 2)[:, :, 0]
    pl_old = par_old % NPT
    pt_in = (par_old // NPT) % TPB
    par_new = jnp.where(pl_old == 0, PB + pt_in, pt_in * CPT + (pl_old - 1))
    par3 = par_new.reshape(NB, TPB * CPT).reshape(NB, 1, PB)

    tvr = tree_vec.reshape(NB, TPB, LAT)

    bz2 = b_z.reshape(1, HID)
    br2 = b_r.reshape(1, HID)
    bh2 = b_h.reshape(1, HID)
    bu2 = b_u.reshape(1, HID)
    bw2 = b_w.reshape(1, HID)
    bo2 = b_o.reshape(1, VOC)
    bs2 = b_s.reshape(1, 1)

    t5 = pl.pallas_call(
        _tables_kernel,
        out_shape=jax.ShapeDtypeStruct((VOC, 5 * HID), f32),
    )(emb, W_z, bz2, W_r, br2, W_h, bh2, U_w, bu2)

    full = lambda shape: pl.BlockSpec(shape, lambda i: (0,) * len(shape))
    parts = pl.pallas_call(
        _main_kernel,
        grid=(NB,),
        in_specs=[
            pl.BlockSpec((1, 1, NBLK), lambda i: (i, 0, 0)),
            pl.BlockSpec((1, 1, PB), lambda i: (i, 0, 0)),
            pl.BlockSpec((1, 1, NBLK), lambda i: (i, 0, 0)),
            pl.BlockSpec((1, TPB, LAT), lambda i: (i, 0, 0)),
            full((VOC, 5 * HID)),
            full((HID, HID)),
            full((HID, HID)),
            full((HID, HID)),
            full((HID, HID)),
            full((LAT, HID)),
            full((1, HID)),
            full((HID, VOC)),
            full((1, VOC)),
            full((HID, HID)),
            full((LAT, HID)),
            full((HID, 1)),
            full((1, 1)),
        ],
        out_specs=pl.BlockSpec((1, 8, 128), lambda i: (i, 0, 0)),
        out_shape=jax.ShapeDtypeStruct((NB, 8, 128), f32),
    )(wid3, par3, ptgt, tvr, t5,
      U_r, W_h[HID:2 * HID], W_z[HID:2 * HID], W_w[0:HID], W_w[HID:HID + LAT],
      bw2, Wo, bo2, U_w[HID:2 * HID], U_w[2 * HID:2 * HID + LAT], Us, bs2)

    sums = jnp.sum(parts[:, 0, 0:4], axis=0)
    q_loss = sums[0] / N_TREES_C
    p_loss = sums[1] / N_TREES_C
    p_acc = sums[2] / (N_TREES_C * NPT)
    q_acc = sums[3] / (N_TREES_C * NPT)
    return (q_loss, p_loss, q_acc, p_acc)


# speedup vs baseline: 1.2512x; 1.2512x over previous
import jax
import jax.numpy as jnp
from jax import lax
from jax.experimental import pallas as pl
from jax.experimental.pallas import tpu as pltpu

N_TREES_C = 1000
NPT = 50
CPT = NPT - 1
HID = 128
LAT = 56
VOC = 780
TPB = 8
NB = N_TREES_C // TPB
NBLK = NPT * TPB
PB = CPT * TPB


def _sig(x):
    return 1.0 / (1.0 + jnp.exp(-x))


def _dot(a, b):
    return jnp.dot(a, b, preferred_element_type=jnp.float32)


def _dot_t(a, b):
    return lax.dot_general(a, b, (((0,), (0,)), ((), ())),
                           preferred_element_type=jnp.float32)


def _tables_kernel(emb_ref, wz_ref, bz_ref, wr_ref, br_ref,
                   wh_ref, bh_ref, uw_ref, bu_ref, out_ref):
    e = emb_ref[...]
    cz = _dot(e, wz_ref[0:HID, :]) + bz_ref[...]
    ch = _dot(e, wh_ref[0:HID, :]) + bh_ref[...]
    m1 = _sig(cz) * jnp.tanh(ch)
    ar = _dot(e, wr_ref[...]) + br_ref[...]
    xu = _dot(e, uw_ref[0:HID, :]) + bu_ref[...]
    out_ref[...] = jnp.concatenate([m1, cz, ch, ar, xu], axis=1)


def _main_kernel(wid_ref, par_ref, pt_ref, tv_ref, t5_ref,
                 ur_ref, wh2_ref, wz2_ref, wqh_ref, wqt_ref, bw_ref, wo_ref,
                 bo_ref, uh_ref, ut_ref, us_ref, bs_ref, out_ref):
    f32 = jnp.float32
    wid = wid_ref[0, 0, :]
    wid_col = wid.reshape(NBLK, 1)
    iota_v = lax.broadcasted_iota(jnp.int32, (NBLK, VOC), 1)
    o_w = (iota_v == wid_col).astype(f32)
    nt = _dot(o_w, t5_ref[...])
    n_m1 = nt[:, 0:128]
    n_cz = nt[:, 128:256]
    n_ch = nt[:, 256:384]
    n_ar = nt[:, 384:512]
    n_xu = nt[:, 512:640]
    n_mz = _dot(n_m1, wz2_ref[...])
    n_br = _dot(n_m1, ur_ref[...])

    par_col = par_ref[0, 0, :].reshape(PB, 1)
    iota_pn = lax.broadcasted_iota(jnp.int32, (PB, NBLK), 1)
    o_par = (iota_pn == par_col).astype(f32)

    ntp = jnp.concatenate([n_m1, n_cz, n_ch, n_ar], axis=1)
    pt4 = _dot(o_par, ntp)
    p_m1 = pt4[:, 0:128]
    p_cz = pt4[:, 128:256]
    p_ch = pt4[:, 256:384]
    p_ar = pt4[:, 384:512]
    p_mz = _dot(p_m1, wz2_ref[...])
    p_br = _dot(p_m1, ur_ref[...])
    c_m1 = n_m1[0:PB, :]
    c_mz = n_mz[0:PB, :]
    c_cz = n_cz[0:PB, :]
    c_ch = n_ch[0:PB, :]
    c_ar = n_ar[0:PB, :]
    c_br = n_br[0:PB, :]

    rm1_e = _sig(c_ar + p_br) * p_m1
    rm1_o = _sig(p_ar + c_br) * c_m1
    g_eo = _dot(jnp.concatenate([rm1_e, rm1_o], axis=0), wh2_ref[...])
    g_e = g_eo[0:PB, :]
    g_o = g_eo[PB:2 * PB, :]

    zpad = jnp.zeros((NBLK - PB, 2 * HID), f32)
    sn2 = (jnp.concatenate([jnp.concatenate([p_m1, g_e], axis=1), zpad],
                           axis=0)
           + _dot_t(o_par, jnp.concatenate([c_m1, g_o], axis=1)))
    sn_m1 = sn2[:, 0:128]
    sn_g = sn2[:, 128:256]
    sn_mz = _dot(sn_m1, wz2_ref[...])
    s_e = _dot(o_par, sn2)
    s_mz_e = _dot(s_e[:, 0:128], wz2_ref[...])

    z2_e = _sig(p_cz + s_mz_e - c_mz)
    t2_e = jnp.tanh(p_ch + s_e[:, 128:256] - g_o)
    s2_e = s_e[:, 0:128] - c_m1
    m2_e = (1.0 - z2_e) * s2_e + z2_e * t2_e
    z2_o = _sig(c_cz + sn_mz[0:PB, :] - p_mz)
    t2_o = jnp.tanh(c_ch + sn_g[0:PB, :] - g_e)
    s2_o = sn_m1[0:PB, :] - p_m1
    m2_o = (1.0 - z2_o) * s2_o + z2_o * t2_o

    zpad1 = jnp.zeros((NBLK - PB, HID), f32)
    h = (jnp.concatenate([m2_e, zpad1], axis=0)
         + _dot_t(o_par, m2_o))

    tv8 = tv_ref[0]
    jj = lax.broadcasted_iota(jnp.int32, (NBLK, TPB), 0)
    t_of = jnp.where(jj < PB, jj // CPT, jj - PB)
    o_t = (t_of == lax.broadcasted_iota(jnp.int32, (NBLK, TPB), 1)).astype(f32)
    tvn = _dot(o_t, tv8)

    qpre = jnp.maximum(_dot(h, wqh_ref[...]) + _dot(tvn, wqt_ref[...])
                       + bw_ref[...], 0.0)
    q = _dot(qpre, wo_ref[...]) + bo_ref[...]
    mx = jnp.max(q, axis=1, keepdims=True)
    lse = jnp.log(jnp.sum(jnp.exp(q - mx), axis=1, keepdims=True)) + mx
    qtrue = jnp.sum(q * o_w, axis=1, keepdims=True)
    qlp = jnp.sum(lse - qtrue)
    idxmat = jnp.where(q == mx, iota_v, VOC)
    amax = jnp.min(idxmat, axis=1, keepdims=True)
    qacc = jnp.sum((amax == wid_col).astype(f32))

    ppre = jnp.maximum(n_xu + _dot(h, uh_ref[...]) + _dot(tvn, ut_ref[...]),
                       0.0)
    p = _dot(ppre, us_ref[...]) + bs_ref[0:1, 0:1]
    ptg = pt_ref[0, 0, :].reshape(NBLK, 1)
    plp = jnp.sum(jnp.maximum(p, 0.0) - p * ptg
                  + jnp.log(1.0 + jnp.exp(-jnp.abs(p))))
    pacc = jnp.sum(((p > 0.0) == (ptg > 0.5)).astype(f32))

    li = lax.broadcasted_iota(jnp.int32, (8, 128), 1)
    si = lax.broadcasted_iota(jnp.int32, (8, 128), 0)
    row0 = si == 0
    res = (jnp.where(row0 & (li == 0), qlp, 0.0)
           + jnp.where(row0 & (li == 1), plp, 0.0)
           + jnp.where(row0 & (li == 2), pacc, 0.0)
           + jnp.where(row0 & (li == 3), qacc, 0.0))
    out_ref[0] = res


def kernel(wid, edge_src, edge_dst, rev_edge, node2tree, p_targets, tree_vec,
           emb, W_z, b_z, W_r, U_r, b_r, W_h, b_h, W_w, b_w, U_w, b_u,
           Wo, b_o, Us, b_s):
    f32 = jnp.float32
    i32 = jnp.int32

    jj = jnp.arange(NBLK, dtype=i32)
    t_in = jnp.where(jj < PB, jj // CPT, jj - PB)
    c_of = jnp.where(jj < PB, jj % CPT + 1, 0)
    old_local = t_in * NPT + c_of

    wid3 = wid.astype(i32).reshape(NB, NBLK)[:, old_local].reshape(NB, 1, NBLK)
    ptgt = (p_targets.astype(f32).reshape(NB, NBLK)[:, old_local]
            .reshape(NB, 1, NBLK))

    par_old = edge_src.astype(i32).reshape(N_TREES_C, CPT, 2)[:, :, 0]
    pl_old = par_old % NPT
    pt_in = (par_old // NPT) % TPB
    par_new = jnp.where(pl_old == 0, PB + pt_in, pt_in * CPT + (pl_old - 1))
    par3 = par_new.reshape(NB, TPB * CPT).reshape(NB, 1, PB)

    tvr = tree_vec.reshape(NB, TPB, LAT)

    bz2 = b_z.reshape(1, HID)
    br2 = b_r.reshape(1, HID)
    bh2 = b_h.reshape(1, HID)
    bu2 = b_u.reshape(1, HID)
    bw2 = b_w.reshape(1, HID)
    bo2 = b_o.reshape(1, VOC)
    bs2 = b_s.reshape(1, 1)

    t5 = pl.pallas_call(
        _tables_kernel,
        out_shape=jax.ShapeDtypeStruct((VOC, 5 * HID), f32),
    )(emb, W_z, bz2, W_r, br2, W_h, bh2, U_w, bu2)

    full = lambda shape: pl.BlockSpec(shape, lambda i: (0,) * len(shape))
    parts = pl.pallas_call(
        _main_kernel,
        grid=(NB,),
        in_specs=[
            pl.BlockSpec((1, 1, NBLK), lambda i: (i, 0, 0)),
            pl.BlockSpec((1, 1, PB), lambda i: (i, 0, 0)),
            pl.BlockSpec((1, 1, NBLK), lambda i: (i, 0, 0)),
            pl.BlockSpec((1, TPB, LAT), lambda i: (i, 0, 0)),
            full((VOC, 5 * HID)),
            full((HID, HID)),
            full((HID, HID)),
            full((HID, HID)),
            full((HID, HID)),
            full((LAT, HID)),
            full((1, HID)),
            full((HID, VOC)),
            full((1, VOC)),
            full((HID, HID)),
            full((LAT, HID)),
            full((HID, 1)),
            full((1, 1)),
        ],
        out_specs=pl.BlockSpec((1, 8, 128), lambda i: (i, 0, 0)),
        out_shape=jax.ShapeDtypeStruct((NB, 8, 128), f32),
    )(wid3, par3, ptgt, tvr, t5,
      U_r, W_h[HID:2 * HID], W_z[HID:2 * HID], W_w[0:HID], W_w[HID:HID + LAT],
      bw2, Wo, bo2, U_w[HID:2 * HID], U_w[2 * HID:2 * HID + LAT], Us, bs2)

    sums = jnp.sum(parts[:, 0, 0:4], axis=0)
    q_loss = sums[0] / N_TREES_C
    p_loss = sums[1] / N_TREES_C
    p_acc = sums[2] / (N_TREES_C * NPT)
    q_acc = sums[3] / (N_TREES_C * NPT)
    return (q_loss, p_loss, q_acc, p_acc)
